# baseline (speedup 1.0000x reference)
import functools

import jax
import jax.numpy as jnp
from jax import lax
from jax.experimental import pallas as pl
from jax.experimental.pallas import tpu as pltpu
from jax.experimental.pallas import tpu_sc as plsc

_BR = 256
_NW = 32
_CH = 8
_NB = 2
_LANES = 16


def _sc_degpart_body(adj_hbm, out_hbm, buf, part, sem0, sem1):
    n = adj_hbm.shape[1]
    rows_w = n // _NW
    n_chunks = rows_w // _CH
    wid = lax.axis_index("s") * 2 + lax.axis_index("c")
    base = wid * rows_w
    sems = (sem0, sem1)

    def copy_in(ci, b):
        return pltpu.make_async_copy(
            adj_hbm.at[0, pl.ds(base + ci * _CH, _CH)], buf.at[b], sems[b])

    for b in range(_NB):
        copy_in(b, b).start()

    for ci in range(n_chunks):
        b = ci % _NB
        copy_in(ci, b).wait()

        def row_body(r, carry):
            def col_body(i, acc):
                for j in range(16):
                    acc = acc + buf[b, r, pl.ds(i * 256 + j * 16, _LANES)]
                return acc
            acc = lax.fori_loop(
                0, n // (16 * _LANES), col_body,
                jnp.zeros((_LANES,), jnp.float32))
            part[ci * _CH + r] = acc
            return carry

        lax.fori_loop(0, _CH, row_body, 0)

        nxt = ci + _NB
        if nxt < n_chunks:
            copy_in(nxt, b).start()

    pltpu.sync_copy(part, out_hbm.at[pl.ds(base, rows_w)])


def _sc_degpart(adj_t):
    n = adj_t.shape[1]
    rows_w = n // _NW
    mesh = plsc.VectorSubcoreMesh(core_axis_name="c", subcore_axis_name="s")
    return functools.partial(
        pl.kernel,
        mesh=mesh,
        out_type=jax.ShapeDtypeStruct((n, _LANES), jnp.float32),
        scratch_types=[
            pltpu.VMEM((_NB, _CH, n), jnp.float32),
            pltpu.VMEM((rows_w, _LANES), jnp.float32),
            pltpu.SemaphoreType.DMA,
            pltpu.SemaphoreType.DMA,
        ],
    )(_sc_degpart_body)(adj_t)


def _prep1_kernel(x_ref, wsum_ref, g_ref):
    g = jnp.dot(x_ref[...], wsum_ref[...], preferred_element_type=jnp.float32)
    g_ref[...] = jnp.concatenate(
        [g, jnp.ones((x_ref.shape[0], 1), jnp.float32)], axis=1)


def _passA_kernel(a1a_ref, a1b_ref, g1_ref, bsum_ref, wo_ref, p1_ref):
    s = pl.program_id(0)
    half = a1a_ref.shape[1] * pl.num_programs(0)
    hid = wo_ref.shape[0]
    g1 = g1_ref[...]

    def half_step(a_ref, base):
        acc = jnp.dot(a_ref[0], g1, preferred_element_type=jnp.float32)
        g1r = g1_ref[pl.ds(base, _BR), :]
        deg1 = acc[:, hid:] + g1r[:, hid:] + 1.0
        gcn = jnp.maximum(
            (acc[:, :hid] + g1r[:, :hid]) / jnp.maximum(deg1, 1e-12)
            + bsum_ref[...], 0.0)
        return jnp.dot(gcn, wo_ref[...], preferred_element_type=jnp.float32)

    p1_ref[0] = half_step(a1a_ref, s * _BR)
    p1_ref[1] = half_step(a1b_ref, half + s * _BR)


def _prep0_kernel(x_ref, wsum_ref, degp_ref, g_ref, d_ref):
    deg0 = jnp.sum(degp_ref[...], axis=1, keepdims=True)
    d = jax.lax.rsqrt(jnp.maximum(deg0 + 1.0, 1e-12))
    d_ref[...] = d
    g_ref[...] = d * jnp.dot(x_ref[...], wsum_ref[...],
                             preferred_element_type=jnp.float32)


def _passB_kernel(a0a_ref, a0b_ref, g0_ref, d_ref, p1_ref, bsum_ref, wo_ref,
                  bout_ref, out_ref):
    s = pl.program_id(0)
    half = a0a_ref.shape[1] * pl.num_programs(0)
    g0 = g0_ref[...]

    def half_step(a_ref, h, base):
        acc = jnp.dot(a_ref[0], g0, preferred_element_type=jnp.float32)
        g0r = g0_ref[pl.ds(base, _BR), :]
        dr = d_ref[pl.ds(base, _BR), :]
        gcn = jnp.maximum(dr * (acc + g0r) + bsum_ref[...], 0.0)
        return jnp.maximum(
            jnp.dot(gcn, wo_ref[...], preferred_element_type=jnp.float32)
            + p1_ref[h, pl.ds(base - h * half, _BR), :] + bout_ref[...], 0.0)

    out_ref[0] = half_step(a0a_ref, 0, s * _BR)
    out_ref[1] = half_step(a0b_ref, 1, half + s * _BR)


def kernel(x, adj_t, W, b, W_out, b_out):
    n, _ = x.shape
    hid = W.shape[-1]
    out_dim = W_out.shape[1]
    n_r = n // _BR
    half_r = n_r // 2

    Wsum = W.sum(axis=1)
    bsum = b.sum(axis=1)[:, None, :]
    wo0, wo1 = W_out[:hid], W_out[hid:]
    bout = b_out[None, :]

    degp = _sc_degpart(adj_t)

    g1 = pl.pallas_call(
        _prep1_kernel,
        out_shape=jax.ShapeDtypeStruct((n, hid + 1), jnp.float32),
    )(x, Wsum[1])

    p1 = pl.pallas_call(
        _passA_kernel,
        grid=(half_r,),
        in_specs=[
            pl.BlockSpec((1, _BR, n), lambda s: (1, s, 0)),
            pl.BlockSpec((1, _BR, n), lambda s: (1, half_r + s, 0)),
            pl.BlockSpec((n, hid + 1), lambda s: (0, 0)),
            pl.BlockSpec((1, hid), lambda s: (0, 0)),
            pl.BlockSpec((hid, out_dim), lambda s: (0, 0)),
        ],
        out_specs=pl.BlockSpec((2, _BR, out_dim), lambda s: (0, s, 0)),
        out_shape=jax.ShapeDtypeStruct((2, n // 2, out_dim), jnp.float32),
    )(adj_t, adj_t, g1, bsum[1], wo1)

    degp, p1 = lax.optimization_barrier((degp, p1))

    g0, dvec = pl.pallas_call(
        _prep0_kernel,
        out_shape=[
            jax.ShapeDtypeStruct((n, hid), jnp.float32),
            jax.ShapeDtypeStruct((n, 1), jnp.float32),
        ],
    )(x, Wsum[0], degp)

    out2 = pl.pallas_call(
        _passB_kernel,
        grid=(half_r,),
        in_specs=[
            pl.BlockSpec((1, _BR, n), lambda s: (0, s, 0)),
            pl.BlockSpec((1, _BR, n), lambda s: (0, half_r + s, 0)),
            pl.BlockSpec((n, hid), lambda s: (0, 0)),
            pl.BlockSpec((n, 1), lambda s: (0, 0)),
            pl.BlockSpec((2, n // 2, out_dim), lambda s: (0, 0, 0)),
            pl.BlockSpec((1, hid), lambda s: (0, 0)),
            pl.BlockSpec((hid, out_dim), lambda s: (0, 0)),
            pl.BlockSpec((1, out_dim), lambda s: (0, 0)),
        ],
        out_specs=pl.BlockSpec((2, _BR, out_dim), lambda s: (0, s, 0)),
        out_shape=jax.ShapeDtypeStruct((2, n // 2, out_dim), jnp.float32),
    )(adj_t, adj_t, g0, dvec, p1, bsum[0], wo0, bout)

    return out2.reshape(n, out_dim)


# speedup vs baseline: 1.1289x; 1.1289x over previous
import jax
import jax.numpy as jnp
from jax.experimental import pallas as pl
from jax.experimental.pallas import tpu as pltpu

_BR = 512


def _prep1_kernel(x_ref, wsum_ref, g_ref):
    g = jnp.dot(x_ref[...], wsum_ref[...], preferred_element_type=jnp.float32)
    g_ref[...] = jnp.concatenate(
        [g, jnp.ones((x_ref.shape[0], 1), jnp.float32)], axis=1)


def _passA_kernel(a0_ref, a1_ref, g1_ref, bsum_ref, wo_ref, p1_ref, deg0_ref):
    s = pl.program_id(0)
    hid = wo_ref.shape[0]
    deg0_ref[...] = jnp.dot(a0_ref[0], g1_ref[:, hid:],
                            preferred_element_type=jnp.float32)
    acc = jnp.dot(a1_ref[0], g1_ref[...], preferred_element_type=jnp.float32)
    g1r = g1_ref[pl.ds(s * _BR, _BR), :]
    deg1 = acc[:, hid:] + g1r[:, hid:] + 1.0
    gcn = jnp.maximum(
        (acc[:, :hid] + g1r[:, :hid]) / jnp.maximum(deg1, 1e-12)
        + bsum_ref[...], 0.0)
    p1_ref[...] = jnp.dot(gcn, wo_ref[...], preferred_element_type=jnp.float32)


def _prep0_kernel(x_ref, wsum_ref, deg0_ref, g_ref, d_ref):
    d = jax.lax.rsqrt(jnp.maximum(deg0_ref[...] + 1.0, 1e-12))
    d_ref[...] = d
    g_ref[...] = d * jnp.dot(x_ref[...], wsum_ref[...],
                             preferred_element_type=jnp.float32)


def _passB_kernel(a0a_ref, a0b_ref, g0_ref, d_ref, p1_ref, bsum_ref, wo_ref,
                  bout_ref, out_ref):
    s = pl.program_id(0)
    half = a0a_ref.shape[1] * pl.num_programs(0)
    g0 = g0_ref[...]

    def half_step(a_ref, base):
        acc = jnp.dot(a_ref[0], g0, preferred_element_type=jnp.float32)
        g0r = g0_ref[pl.ds(base, _BR), :]
        dr = d_ref[pl.ds(base, _BR), :]
        gcn = jnp.maximum(dr * (acc + g0r) + bsum_ref[...], 0.0)
        return jnp.maximum(
            jnp.dot(gcn, wo_ref[...], preferred_element_type=jnp.float32)
            + p1_ref[pl.ds(base, _BR), :] + bout_ref[...], 0.0)

    out_ref[0] = half_step(a0a_ref, s * _BR)
    out_ref[1] = half_step(a0b_ref, half + s * _BR)


def kernel(x, adj_t, W, b, W_out, b_out):
    n, _ = x.shape
    hid = W.shape[-1]
    out_dim = W_out.shape[1]
    n_r = n // _BR
    half_r = n_r // 2

    Wsum = W.sum(axis=1)
    bsum = b.sum(axis=1)[:, None, :]
    wo0, wo1 = W_out[:hid], W_out[hid:]
    bout = b_out[None, :]

    g1 = pl.pallas_call(
        _prep1_kernel,
        out_shape=jax.ShapeDtypeStruct((n, hid + 1), jnp.float32),
    )(x, Wsum[1])

    p1, deg0 = pl.pallas_call(
        _passA_kernel,
        grid=(n_r,),
        in_specs=[
            pl.BlockSpec((1, _BR, n), lambda s: (0, s, 0)),
            pl.BlockSpec((1, _BR, n), lambda s: (1, s, 0)),
            pl.BlockSpec((n, hid + 1), lambda s: (0, 0)),
            pl.BlockSpec((1, hid), lambda s: (0, 0)),
            pl.BlockSpec((hid, out_dim), lambda s: (0, 0)),
        ],
        out_specs=[
            pl.BlockSpec((_BR, out_dim), lambda s: (s, 0)),
            pl.BlockSpec((_BR, 1), lambda s: (s, 0)),
        ],
        out_shape=[
            jax.ShapeDtypeStruct((n, out_dim), jnp.float32),
            jax.ShapeDtypeStruct((n, 1), jnp.float32),
        ],
    )(adj_t, adj_t, g1, bsum[1], wo1)

    g0, dvec = pl.pallas_call(
        _prep0_kernel,
        out_shape=[
            jax.ShapeDtypeStruct((n, hid), jnp.float32),
            jax.ShapeDtypeStruct((n, 1), jnp.float32),
        ],
    )(x, Wsum[0], deg0)

    out2 = pl.pallas_call(
        _passB_kernel,
        grid=(half_r,),
        in_specs=[
            pl.BlockSpec((1, _BR, n), lambda s: (0, s, 0)),
            pl.BlockSpec((1, _BR, n), lambda s: (0, half_r + s, 0)),
            pl.BlockSpec((n, hid), lambda s: (0, 0)),
            pl.BlockSpec((n, 1), lambda s: (0, 0)),
            pl.BlockSpec((n, out_dim), lambda s: (0, 0)),
            pl.BlockSpec((1, hid), lambda s: (0, 0)),
            pl.BlockSpec((hid, out_dim), lambda s: (0, 0)),
            pl.BlockSpec((1, out_dim), lambda s: (0, 0)),
        ],
        out_specs=pl.BlockSpec((2, _BR, out_dim), lambda s: (0, s, 0)),
        out_shape=jax.ShapeDtypeStruct((2, n // 2, out_dim), jnp.float32),
    )(adj_t, adj_t, g0, dvec, p1, bsum[0], wo0, bout)

    return out2.reshape(n, out_dim)
